# Initial kernel scaffold; baseline (speedup 1.0000x reference)
import functools

import jax
import jax.numpy as jnp
from jax import lax
from jax.experimental import pallas as pl
from jax.experimental.pallas import tpu as pltpu
from jax.experimental.pallas import tpu_sc as plsc

VOCAB = 1000000
D = 32
B = 16384
L = 200
NCLS = 4

NC, NS = 2, 16
NW = NC * NS
BAGS_PER_W = B // NW
NB = 16
NCHUNK = BAGS_PER_W // NB
IDX_ROWS = NB * L // 128


def _sc_bagsum(text2d, emb_table):
    mesh = plsc.VectorSubcoreMesh(core_axis_name="c", subcore_axis_name="s")

    @functools.partial(
        pl.kernel,
        mesh=mesh,
        out_type=jax.ShapeDtypeStruct((B, D), jnp.float32),
        scratch_types=[
            pltpu.VMEM((IDX_ROWS, 128), jnp.int32),
            pltpu.VMEM((NB * L, D), jnp.float32),
            pltpu.VMEM((NB, D), jnp.float32),
            pltpu.SemaphoreType.DMA,
        ],
    )
    def k(text_hbm, emb_hbm, out_hbm, idx_v, rows_v, acc_v, sem):
        wid = lax.axis_index("s") * NC + lax.axis_index("c")

        @pl.loop(0, NCHUNK)
        def _(g):
            bag0 = wid * BAGS_PER_W + g * NB
            row0 = wid * (BAGS_PER_W * L // 128) + g * IDX_ROWS
            pltpu.sync_copy(text_hbm.at[pl.ds(row0, IDX_ROWS)], idx_v)
            copies = [
                pltpu.async_copy(
                    emb_hbm.at[idx_v.at[kk]],
                    rows_v.at[pl.ds(kk * 128, 128)],
                    sem,
                )
                for kk in range(IDX_ROWS)
            ]
            for c in copies:
                c.wait()
            for i in range(NB):
                def body(j, carry):
                    lo, hi = carry
                    return (lo + rows_v[i * L + j, pl.ds(0, 16)],
                            hi + rows_v[i * L + j, pl.ds(16, 16)])
                zero = jnp.zeros((16,), jnp.float32)
                lo, hi = lax.fori_loop(0, L, body, (zero, zero))
                acc_v[i, pl.ds(0, 16)] = lo
                acc_v[i, pl.ds(16, 16)] = hi
            pltpu.sync_copy(acc_v, out_hbm.at[pl.ds(bag0, NB)])

    return k(text2d, emb_table)


def _tc_linear(sums, w, b2):
    def body(x_ref, w_ref, b_ref, o_ref):
        o_ref[...] = jnp.dot(
            x_ref[...], w_ref[...],
            preferred_element_type=jnp.float32,
            precision=lax.Precision.HIGHEST,
        ) + b_ref[...]

    blk = 2048
    return pl.pallas_call(
        body,
        grid=(B // blk,),
        in_specs=[
            pl.BlockSpec((blk, D), lambda i: (i, 0)),
            pl.BlockSpec((D, NCLS), lambda i: (0, 0)),
            pl.BlockSpec((1, NCLS), lambda i: (0, 0)),
        ],
        out_specs=pl.BlockSpec((blk, NCLS), lambda i: (i, 0)),
        out_shape=jax.ShapeDtypeStruct((B, NCLS), jnp.float32),
    )(sums, w, b2)


def kernel(text, emb_table, fc_w, fc_b):
    text2d = text.astype(jnp.int32).reshape(B * L // 128, 128)
    sums = _sc_bagsum(text2d, emb_table)
    w = (fc_w.T / jnp.float32(L)).astype(jnp.float32)
    b2 = fc_b.reshape(1, NCLS).astype(jnp.float32)
    return _tc_linear(sums, w, b2)


# speedup vs baseline: 10.9815x; 10.9815x over previous
import functools

import jax
import jax.numpy as jnp
from jax import lax
from jax.experimental import pallas as pl
from jax.experimental.pallas import tpu as pltpu
from jax.experimental.pallas import tpu_sc as plsc

VOCAB = 1000000
D = 32
B = 16384
L = 200
NCLS = 4

NC, NS = 2, 16
NW = NC * NS
BAGS_PER_W = B // NW
NB = 16
NCHUNK = BAGS_PER_W // NB
IDX_ROWS = NB * L // 128


def _sc_bagsum(text_flat, emb_table):
    mesh = plsc.VectorSubcoreMesh(core_axis_name="c", subcore_axis_name="s")

    @functools.partial(
        pl.kernel,
        mesh=mesh,
        out_type=jax.ShapeDtypeStruct((B, D), jnp.float32),
        scratch_types=[
            pltpu.VMEM((NB * L,), jnp.int32),
            pltpu.VMEM((NB * L, D), jnp.float32),
            pltpu.VMEM((NB, D), jnp.float32),
            pltpu.SemaphoreType.DMA,
        ],
        compiler_params=pltpu.CompilerParams(use_tc_tiling_on_sc=False),
    )
    def k(text_hbm, emb_hbm, out_hbm, idx_v, rows_v, acc_v, sem):
        wid = lax.axis_index("s") * NC + lax.axis_index("c")

        @pl.loop(0, NCHUNK)
        def _(g):
            bag0 = wid * BAGS_PER_W + g * NB
            pltpu.sync_copy(text_hbm.at[pl.ds(bag0 * L, NB * L)], idx_v)
            copies = [
                pltpu.async_copy(
                    emb_hbm.at[idx_v.at[pl.ds(kk * 128, 128)]],
                    rows_v.at[pl.ds(kk * 128, 128)],
                    sem,
                )
                for kk in range(IDX_ROWS)
            ]
            for c in copies:
                c.wait()
            for i in range(NB):
                def body(j, carry):
                    lo, hi = carry
                    return (lo + rows_v[i * L + j, pl.ds(0, 16)],
                            hi + rows_v[i * L + j, pl.ds(16, 16)])
                zero = jnp.zeros((16,), jnp.float32)
                lo, hi = lax.fori_loop(0, L, body, (zero, zero))
                acc_v[i, pl.ds(0, 16)] = lo
                acc_v[i, pl.ds(16, 16)] = hi
            pltpu.sync_copy(acc_v, out_hbm.at[pl.ds(bag0, NB)])

    return k(text_flat, emb_table)


def _tc_linear(sums, w, b2):
    def body(x_ref, w_ref, b_ref, o_ref):
        o_ref[...] = jnp.dot(
            x_ref[...], w_ref[...],
            preferred_element_type=jnp.float32,
            precision=lax.Precision.HIGHEST,
        ) + b_ref[...]

    blk = 2048
    return pl.pallas_call(
        body,
        grid=(B // blk,),
        in_specs=[
            pl.BlockSpec((blk, D), lambda i: (i, 0)),
            pl.BlockSpec((D, NCLS), lambda i: (0, 0)),
            pl.BlockSpec((1, NCLS), lambda i: (0, 0)),
        ],
        out_specs=pl.BlockSpec((blk, NCLS), lambda i: (i, 0)),
        out_shape=jax.ShapeDtypeStruct((B, NCLS), jnp.float32),
    )(sums, w, b2)


def kernel(text, emb_table, fc_w, fc_b):
    text_flat = text.astype(jnp.int32).reshape(B * L)
    sums = _sc_bagsum(text_flat, emb_table)
    w = (fc_w.T / jnp.float32(L)).astype(jnp.float32)
    b2 = fc_b.reshape(1, NCLS).astype(jnp.float32)
    return _tc_linear(sums, w, b2)
